# baseline (speedup 1.0000x reference)
import jax
import jax.numpy as jnp
from jax import lax
from jax.experimental import pallas as pl
from jax.experimental.pallas import tpu as pltpu
from jax.experimental.pallas import tpu_sc as plsc

NUM_CORES = 2
NUM_SUBCORES = 16
LANES = 16
NUM_WORKERS = NUM_CORES * NUM_SUBCORES

BATCH = 16384
EMB_DIM = 64
NROW = 1000000
RANGE = 31232
CHUNK = 256
N_CHUNK = RANGE // CHUNK
TAIL0 = NUM_WORKERS * RANGE
TAIL_MAIN = 512
TAIL_PATCH = 128
TAILP0 = NROW - TAIL_PATCH
LIST_CAP = 2 * BATCH + 2 * LANES
IDXC = 2048
B_PER_W = BATCH // NUM_WORKERS
RING = 32
GROUPS = B_PER_W // LANES
NBINS_PAD = 144


def _scan_body(bu_hbm, bpi_hbm, bni_hbm, ut_hbm, it_hbm, utail_hbm, itail_hbm,
               du_hbm, di_hbm, dj_hbm,
               lst, lst2, bufa, bufb, bufc, tailbuf, idxc, hist, offs,
               stag, ssem, hsem):
    wid = lax.axis_index("s") * NUM_CORES + lax.axis_index("c")
    lo = wid * RANGE
    is_last = wid == NUM_WORKERS - 1
    hi = jnp.where(is_last, NROW, lo + RANGE)

    def build(arr_hbm, tag):
        def chunk_body(ci, cnt):
            pltpu.sync_copy(arr_hbm.at[pl.ds(ci * IDXC, IDXC)], idxc)

            def vec_body(v, cnt):
                r = idxc[pl.ds(v * LANES, LANES)]
                m = (r >= lo) & (r < hi)
                b = ci * IDXC + v * LANES + lax.iota(jnp.int32, LANES)
                entry = ((r - lo) << 16) | (tag << 14) | b
                plsc.store_compressed(lst.at[pl.ds(cnt, LANES)], entry,
                                      mask=m)
                n = plsc.all_reduce_population_count(m)
                return cnt + n[0]

            return lax.fori_loop(0, IDXC // LANES, vec_body, cnt,
                                 unroll=False)

        return chunk_body

    def build_list(arrs):
        cnt = 0
        for arr, tag in arrs:
            cnt = lax.fori_loop(0, BATCH // IDXC, build(arr, tag), cnt,
                                unroll=False)
        return cnt

    def drain_one():
        pltpu.make_async_copy(du_hbm.at[pl.ds(0, EMB_DIM)],
                              stag.at[0], hsem).wait()

    lanes_iota = lax.iota(jnp.int32, LANES)
    lane0 = lanes_iota == 0

    def bucketize(cnt):
        zero = jnp.zeros((LANES,), jnp.int32)
        for z in range(NBINS_PAD // LANES):
            hist[pl.ds(z * LANES, LANES)] = zero

        def count_body(p, carry):
            el = lst[pl.ds(p, LANES)][0]
            bn = el >> 24
            hv = hist[pl.ds(bn, LANES)]
            hist[pl.ds(bn, LANES)] = jnp.where(lane0, hv + 1, hv)
            return carry

        lax.fori_loop(0, cnt, count_body, 0, unroll=False)

        tot = 0
        for g in range(NBINS_PAD // LANES):
            v = hist[pl.ds(g * LANES, LANES)]
            c = plsc.cumsum(v)
            excl = tot + c - v
            offs[pl.ds(g * LANES, LANES)] = excl
            hist[pl.ds(g * LANES, LANES)] = excl
            tot = tot + c[LANES - 1]

        def place_body(p, carry):
            ev = lst[pl.ds(p, LANES)]
            el = ev[0]
            bn = el >> 24
            ov = offs[pl.ds(bn, LANES)]
            o = ov[0]
            offs[pl.ds(bn, LANES)] = jnp.where(lane0, ov + 1, ov)
            plsc.store_compressed(lst2.at[pl.ds(o, LANES)], ev, mask=lane0)
            return carry

        lax.fori_loop(0, cnt, place_body, 0, unroll=False)

    def process(buf, c0, pass_user, fired0, bin_override=None):
        bn = c0 >> 8 if bin_override is None else bin_override
        s = hist[pl.ds(bn, LANES)][0]
        epos = offs[pl.ds(bn, LANES)][0]

        def ent_body(p, fired):
            el = lst2[pl.ds(p, LANES)][0]
            slot = fired & (RING - 1)

            @pl.when(fired >= RING)
            def _():
                drain_one()

            cl = (el >> 16) - c0
            b = el & (BATCH - 1)
            clv = jnp.full((LANES,), cl, jnp.int32)
            for v4 in range(EMB_DIM // LANES):
                g = plsc.load_gather(buf, [lanes_iota + v4 * LANES, clv])
                stag[slot, pl.ds(v4 * LANES, LANES)] = g
            dst = pl.ds(b * EMB_DIM, EMB_DIM)
            src = stag.at[slot]
            if pass_user:
                pltpu.async_copy(src, du_hbm.at[dst], hsem)
            else:
                tl = (el >> 14) & 3
                @pl.when(tl == 1)
                def _():
                    pltpu.async_copy(src, di_hbm.at[dst], hsem)
                @pl.when(tl == 2)
                def _():
                    pltpu.async_copy(src, dj_hbm.at[dst], hsem)

            return fired + 1

        return lax.fori_loop(s, epos, ent_body, fired0, unroll=False)

    def stream_range(tab_hbm, tail_hbm, pass_user):
        def start(k, buf):
            for c8 in range(EMB_DIM // 8):
                pltpu.async_copy(
                    tab_hbm.at[pl.ds(c8 * 8, 8), pl.ds(lo + k * CHUNK, CHUNK)],
                    buf.at[pl.ds(c8 * 8, 8)], ssem)

        def start_if(k, buf):
            @pl.when(k < N_CHUNK)
            def _():
                start(k, buf)

        def wait(buf):
            pltpu.make_async_copy(tab_hbm.at[:, pl.ds(0, CHUNK)], buf,
                                  ssem).wait()

        start(0, bufa)
        start(1, bufb)

        def tri_body(p, fired):
            k = p * 3
            wait(bufa)
            start_if(k + 2, bufc)
            fired = process(bufa, k * CHUNK, pass_user, fired)
            wait(bufb)
            start_if(k + 3, bufa)
            fired = process(bufb, (k + 1) * CHUNK, pass_user, fired)
            wait(bufc)
            start_if(k + 4, bufb)
            fired = process(bufc, (k + 2) * CHUNK, pass_user, fired)
            return fired

        fired = lax.fori_loop(0, N_CHUNK // 3, tri_body, 0, unroll=False)
        wait(bufa)
        fired = process(bufa, (N_CHUNK - 2) * CHUNK, pass_user, fired)
        wait(bufb)
        fired = process(bufb, (N_CHUNK - 1) * CHUNK, pass_user, fired)

        def drain_n(n):
            def drain_body(d, carry):
                drain_one()
                return carry

            lax.fori_loop(0, jnp.minimum(n, RING), drain_body, 0,
                          unroll=False)

        @pl.when(is_last)
        def _():
            f2 = fired
            for t in range(TAIL_MAIN // CHUNK):
                pltpu.sync_copy(
                    tab_hbm.at[:, pl.ds(TAIL0 + t * CHUNK, CHUNK)], bufa)
                f2 = process(bufa, TAIL0 - lo + t * CHUNK, pass_user, f2)
            pltpu.sync_copy(tail_hbm, tailbuf)
            f3 = process(tailbuf, TAILP0 - lo, pass_user, f2,
                         bin_override=(TAIL0 - lo + TAIL_MAIN) >> 8)
            drain_n(f3)

        @pl.when(jnp.logical_not(is_last))
        def _():
            drain_n(fired)

    cnt_u = build_list(((bu_hbm, 0),))
    bucketize(cnt_u)
    stream_range(ut_hbm, utail_hbm, True)
    cnt_i = build_list(((bpi_hbm, 1), (bni_hbm, 2)))
    bucketize(cnt_i)
    stream_range(it_hbm, itail_hbm, False)


def _dot_body(du_hbm, di_hbm, dj_hbm, pos_hbm, neg_hbm,
              ru, ri, rj, accp_s, accn_s, pos_v, neg_v):
    wid = lax.axis_index("s") * NUM_CORES + lax.axis_index("c")
    base = wid * B_PER_W
    nwords = B_PER_W * EMB_DIM
    pltpu.sync_copy(du_hbm.at[pl.ds(base * EMB_DIM, nwords)], ru)
    pltpu.sync_copy(di_hbm.at[pl.ds(base * EMB_DIM, nwords)], ri)
    pltpu.sync_copy(dj_hbm.at[pl.ds(base * EMB_DIM, nwords)], rj)
    lanes_iota = lax.iota(jnp.int32, LANES)

    def group_body(g, carry):
        for row_l in range(LANES):
            off = (g * LANES + row_l) * EMB_DIM
            accp = jnp.zeros((LANES,), jnp.float32)
            accn = jnp.zeros((LANES,), jnp.float32)
            for v in range(EMB_DIM // LANES):
                sl = pl.ds(off + v * LANES, LANES)
                u = ru[sl]
                iv = ri[sl]
                jv = rj[sl]
                accp = accp + u * iv
                accn = accn + u * jv
            accp_s[row_l] = accp
            accn_s[row_l] = accn
        sump = jnp.zeros((LANES,), jnp.float32)
        sumn = jnp.zeros((LANES,), jnp.float32)
        for l in range(LANES):
            col = jnp.full((LANES,), l, jnp.int32)
            sump = sump + plsc.load_gather(accp_s, [lanes_iota, col])
            sumn = sumn + plsc.load_gather(accn_s, [lanes_iota, col])
        out = pl.ds(g * LANES, LANES)
        pos_v[out] = sump
        neg_v[out] = sumn
        return carry

    lax.fori_loop(0, GROUPS, group_body, 0, unroll=False)
    pltpu.sync_copy(pos_v, pos_hbm.at[pl.ds(base, B_PER_W)])
    pltpu.sync_copy(neg_v, neg_hbm.at[pl.ds(base, B_PER_W)])


@jax.jit
def _bpr_scores(batch_user, batch_pos_item, batch_neg_item,
                user_emb_t, item_emb_t, user_tail, item_tail):
    mesh = plsc.VectorSubcoreMesh(core_axis_name="c", subcore_axis_name="s",
                                  num_cores=NUM_CORES,
                                  num_subcores=NUM_SUBCORES)
    cparams = pltpu.CompilerParams(needs_layout_passes=False,
                                   use_tc_tiling_on_sc=True)
    scan = pl.kernel(
        _scan_body,
        out_type=[jax.ShapeDtypeStruct((BATCH * EMB_DIM,), jnp.float32)] * 3,
        mesh=mesh,
        compiler_params=cparams,
        scratch_types=[
            pltpu.VMEM((LIST_CAP,), jnp.int32),
            pltpu.VMEM((LIST_CAP,), jnp.int32),
            pltpu.VMEM((EMB_DIM, CHUNK), jnp.float32),
            pltpu.VMEM((EMB_DIM, CHUNK), jnp.float32),
            pltpu.VMEM((EMB_DIM, CHUNK), jnp.float32),
            pltpu.VMEM((EMB_DIM, TAIL_PATCH), jnp.float32),
            pltpu.VMEM((IDXC,), jnp.int32),
            pltpu.VMEM((NBINS_PAD,), jnp.int32),
            pltpu.VMEM((NBINS_PAD,), jnp.int32),
            pltpu.VMEM((RING, EMB_DIM), jnp.float32),
            pltpu.SemaphoreType.DMA,
            pltpu.SemaphoreType.DMA,
        ],
    )
    du, di, dj = scan(batch_user, batch_pos_item, batch_neg_item,
                      user_emb_t, item_emb_t, user_tail, item_tail)
    dot = pl.kernel(
        _dot_body,
        out_type=[jax.ShapeDtypeStruct((BATCH,), jnp.float32)] * 2,
        mesh=mesh,
        compiler_params=cparams,
        scratch_types=[
            pltpu.VMEM((B_PER_W * EMB_DIM,), jnp.float32),
            pltpu.VMEM((B_PER_W * EMB_DIM,), jnp.float32),
            pltpu.VMEM((B_PER_W * EMB_DIM,), jnp.float32),
            pltpu.VMEM((LANES, LANES), jnp.float32),
            pltpu.VMEM((LANES, LANES), jnp.float32),
            pltpu.VMEM((B_PER_W,), jnp.float32),
            pltpu.VMEM((B_PER_W,), jnp.float32),
        ],
    )
    return dot(du, di, dj)


def kernel(batch_user, batch_pos_item, batch_neg_item, user_emb, item_emb):
    ut = user_emb.T
    it = item_emb.T
    pos, neg = _bpr_scores(batch_user.astype(jnp.int32),
                           batch_pos_item.astype(jnp.int32),
                           batch_neg_item.astype(jnp.int32),
                           ut, it,
                           ut[:, TAILP0:],
                           it[:, TAILP0:])
    return (pos[:, None], neg[:, None])


# speedup vs baseline: 1.0081x; 1.0081x over previous
import jax
import jax.numpy as jnp
from jax import lax
from jax.experimental import pallas as pl
from jax.experimental.pallas import tpu as pltpu
from jax.experimental.pallas import tpu_sc as plsc

NUM_CORES = 2
NUM_SUBCORES = 16
LANES = 16
NUM_WORKERS = NUM_CORES * NUM_SUBCORES

BATCH = 16384
EMB_DIM = 64
NROW = 1000000
RANGE = 31232
CHUNK = 256
N_CHUNK = RANGE // CHUNK
TAIL0 = NUM_WORKERS * RANGE
TAIL_MAIN = 512
TAIL_PATCH = 128
TAILP0 = NROW - TAIL_PATCH
LIST_CAP = 2 * BATCH + 2 * LANES
IDXC = 2048
B_PER_W = BATCH // NUM_WORKERS
RING = 32
GROUPS = B_PER_W // LANES
NBINS_PAD = 144


def _scan_body(bu_hbm, bpi_hbm, bni_hbm, ut_hbm, it_hbm, utail_hbm, itail_hbm,
               du_hbm, di_hbm, dj_hbm,
               lst, lst2, bufa, bufb, bufc, tailbuf, idxc, hist, offs,
               stag, ssem, hsem):
    wid = lax.axis_index("s") * NUM_CORES + lax.axis_index("c")
    lo = wid * RANGE
    is_last = wid == NUM_WORKERS - 1
    hi = jnp.where(is_last, NROW, lo + RANGE)

    def build(arr_hbm, tag):
        def chunk_body(ci, cnt):
            pltpu.sync_copy(arr_hbm.at[pl.ds(ci * IDXC, IDXC)], idxc)

            def vec_body(v, cnt):
                r = idxc[pl.ds(v * LANES, LANES)]
                m = (r >= lo) & (r < hi)
                b = ci * IDXC + v * LANES + lax.iota(jnp.int32, LANES)
                entry = ((r - lo) << 16) | (tag << 14) | b
                plsc.store_compressed(lst.at[pl.ds(cnt, LANES)], entry,
                                      mask=m)
                n = plsc.all_reduce_population_count(m)
                return cnt + n[0]

            return lax.fori_loop(0, IDXC // LANES, vec_body, cnt,
                                 unroll=False)

        return chunk_body

    def build_list(arrs):
        cnt = 0
        for arr, tag in arrs:
            cnt = lax.fori_loop(0, BATCH // IDXC, build(arr, tag), cnt,
                                unroll=False)
        return cnt

    def drain_one():
        pltpu.make_async_copy(du_hbm.at[pl.ds(0, EMB_DIM)],
                              stag.at[0], hsem).wait()

    lanes_iota = lax.iota(jnp.int32, LANES)
    lane0 = lanes_iota == 0

    def bucketize(cnt):
        zero = jnp.zeros((LANES,), jnp.int32)
        for z in range(NBINS_PAD // LANES):
            hist[pl.ds(z * LANES, LANES)] = zero

        def count_body(p, carry):
            el = lst[pl.ds(p, LANES)][0]
            bn = el >> 24
            hv = hist[pl.ds(bn, LANES)]
            hist[pl.ds(bn, LANES)] = jnp.where(lane0, hv + 1, hv)
            return carry

        lax.fori_loop(0, cnt, count_body, 0, unroll=False)

        tot = 0
        for g in range(NBINS_PAD // LANES):
            v = hist[pl.ds(g * LANES, LANES)]
            c = plsc.cumsum(v)
            excl = tot + c - v
            offs[pl.ds(g * LANES, LANES)] = excl
            hist[pl.ds(g * LANES, LANES)] = excl
            tot = tot + c[LANES - 1]

        def place_body(p, carry):
            ev = lst[pl.ds(p, LANES)]
            el = ev[0]
            bn = el >> 24
            ov = offs[pl.ds(bn, LANES)]
            o = ov[0]
            offs[pl.ds(bn, LANES)] = jnp.where(lane0, ov + 1, ov)
            plsc.store_compressed(lst2.at[pl.ds(o, LANES)], ev, mask=lane0)
            return carry

        lax.fori_loop(0, cnt, place_body, 0, unroll=False)

    def process(buf, c0, pass_user, fired0, bin_override=None):
        bn = c0 >> 8 if bin_override is None else bin_override
        s = hist[pl.ds(bn, LANES)][0]
        epos = offs[pl.ds(bn, LANES)][0]

        def ent_body(p, fired):
            el = lst2[pl.ds(p, LANES)][0]
            slot = fired & (RING - 1)

            @pl.when(fired >= RING)
            def _():
                drain_one()

            cl = (el >> 16) - c0
            b = el & (BATCH - 1)
            clv = jnp.full((LANES,), cl, jnp.int32)
            for v4 in range(EMB_DIM // LANES):
                g = plsc.load_gather(buf, [lanes_iota + v4 * LANES, clv])
                stag[slot, pl.ds(v4 * LANES, LANES)] = g
            dst = pl.ds(b * EMB_DIM, EMB_DIM)
            src = stag.at[slot]
            if pass_user:
                pltpu.async_copy(src, du_hbm.at[dst], hsem)
            else:
                tl = (el >> 14) & 3
                @pl.when(tl == 1)
                def _():
                    pltpu.async_copy(src, di_hbm.at[dst], hsem)
                @pl.when(tl == 2)
                def _():
                    pltpu.async_copy(src, dj_hbm.at[dst], hsem)

            return fired + 1

        return lax.fori_loop(s, epos, ent_body, fired0, unroll=False)

    def prestart(tab_hbm):
        for c8 in range(EMB_DIM // 8):
            pltpu.async_copy(
                tab_hbm.at[pl.ds(c8 * 8, 8), pl.ds(lo, CHUNK)],
                bufa.at[pl.ds(c8 * 8, 8)], ssem)
        for c8 in range(EMB_DIM // 8):
            pltpu.async_copy(
                tab_hbm.at[pl.ds(c8 * 8, 8), pl.ds(lo + CHUNK, CHUNK)],
                bufb.at[pl.ds(c8 * 8, 8)], ssem)

    def stream_range(tab_hbm, tail_hbm, pass_user):
        def start(k, buf):
            for c8 in range(EMB_DIM // 8):
                pltpu.async_copy(
                    tab_hbm.at[pl.ds(c8 * 8, 8), pl.ds(lo + k * CHUNK, CHUNK)],
                    buf.at[pl.ds(c8 * 8, 8)], ssem)

        def start_if(k, buf):
            @pl.when(k < N_CHUNK)
            def _():
                start(k, buf)

        def wait(buf):
            pltpu.make_async_copy(tab_hbm.at[:, pl.ds(0, CHUNK)], buf,
                                  ssem).wait()

        def tri_body(p, fired):
            k = p * 3
            wait(bufa)
            start_if(k + 2, bufc)
            fired = process(bufa, k * CHUNK, pass_user, fired)
            wait(bufb)
            start_if(k + 3, bufa)
            fired = process(bufb, (k + 1) * CHUNK, pass_user, fired)
            wait(bufc)
            start_if(k + 4, bufb)
            fired = process(bufc, (k + 2) * CHUNK, pass_user, fired)
            return fired

        fired = lax.fori_loop(0, N_CHUNK // 3, tri_body, 0, unroll=False)
        wait(bufa)
        fired = process(bufa, (N_CHUNK - 2) * CHUNK, pass_user, fired)
        wait(bufb)
        fired = process(bufb, (N_CHUNK - 1) * CHUNK, pass_user, fired)

        def drain_n(n):
            def drain_body(d, carry):
                drain_one()
                return carry

            lax.fori_loop(0, jnp.minimum(n, RING), drain_body, 0,
                          unroll=False)

        @pl.when(is_last)
        def _():
            f2 = fired
            for t in range(TAIL_MAIN // CHUNK):
                pltpu.sync_copy(
                    tab_hbm.at[:, pl.ds(TAIL0 + t * CHUNK, CHUNK)], bufa)
                f2 = process(bufa, TAIL0 - lo + t * CHUNK, pass_user, f2)
            pltpu.sync_copy(tail_hbm, tailbuf)
            f3 = process(tailbuf, TAILP0 - lo, pass_user, f2,
                         bin_override=(TAIL0 - lo + TAIL_MAIN) >> 8)
            drain_n(f3)

        @pl.when(jnp.logical_not(is_last))
        def _():
            drain_n(fired)

    prestart(ut_hbm)
    cnt_u = build_list(((bu_hbm, 0),))
    bucketize(cnt_u)
    stream_range(ut_hbm, utail_hbm, True)
    prestart(it_hbm)
    cnt_i = build_list(((bpi_hbm, 1), (bni_hbm, 2)))
    bucketize(cnt_i)
    stream_range(it_hbm, itail_hbm, False)


def _dot_body(du_hbm, di_hbm, dj_hbm, pos_hbm, neg_hbm,
              ru, ri, rj, accp_s, accn_s, pos_v, neg_v):
    wid = lax.axis_index("s") * NUM_CORES + lax.axis_index("c")
    base = wid * B_PER_W
    nwords = B_PER_W * EMB_DIM
    pltpu.sync_copy(du_hbm.at[pl.ds(base * EMB_DIM, nwords)], ru)
    pltpu.sync_copy(di_hbm.at[pl.ds(base * EMB_DIM, nwords)], ri)
    pltpu.sync_copy(dj_hbm.at[pl.ds(base * EMB_DIM, nwords)], rj)
    lanes_iota = lax.iota(jnp.int32, LANES)

    def group_body(g, carry):
        for row_l in range(LANES):
            off = (g * LANES + row_l) * EMB_DIM
            accp = jnp.zeros((LANES,), jnp.float32)
            accn = jnp.zeros((LANES,), jnp.float32)
            for v in range(EMB_DIM // LANES):
                sl = pl.ds(off + v * LANES, LANES)
                u = ru[sl]
                iv = ri[sl]
                jv = rj[sl]
                accp = accp + u * iv
                accn = accn + u * jv
            accp_s[row_l] = accp
            accn_s[row_l] = accn
        sump = jnp.zeros((LANES,), jnp.float32)
        sumn = jnp.zeros((LANES,), jnp.float32)
        for l in range(LANES):
            col = jnp.full((LANES,), l, jnp.int32)
            sump = sump + plsc.load_gather(accp_s, [lanes_iota, col])
            sumn = sumn + plsc.load_gather(accn_s, [lanes_iota, col])
        out = pl.ds(g * LANES, LANES)
        pos_v[out] = sump
        neg_v[out] = sumn
        return carry

    lax.fori_loop(0, GROUPS, group_body, 0, unroll=False)
    pltpu.sync_copy(pos_v, pos_hbm.at[pl.ds(base, B_PER_W)])
    pltpu.sync_copy(neg_v, neg_hbm.at[pl.ds(base, B_PER_W)])


@jax.jit
def _bpr_scores(batch_user, batch_pos_item, batch_neg_item,
                user_emb_t, item_emb_t, user_tail, item_tail):
    mesh = plsc.VectorSubcoreMesh(core_axis_name="c", subcore_axis_name="s",
                                  num_cores=NUM_CORES,
                                  num_subcores=NUM_SUBCORES)
    cparams = pltpu.CompilerParams(needs_layout_passes=False,
                                   use_tc_tiling_on_sc=True)
    scan = pl.kernel(
        _scan_body,
        out_type=[jax.ShapeDtypeStruct((BATCH * EMB_DIM,), jnp.float32)] * 3,
        mesh=mesh,
        compiler_params=cparams,
        scratch_types=[
            pltpu.VMEM((LIST_CAP,), jnp.int32),
            pltpu.VMEM((LIST_CAP,), jnp.int32),
            pltpu.VMEM((EMB_DIM, CHUNK), jnp.float32),
            pltpu.VMEM((EMB_DIM, CHUNK), jnp.float32),
            pltpu.VMEM((EMB_DIM, CHUNK), jnp.float32),
            pltpu.VMEM((EMB_DIM, TAIL_PATCH), jnp.float32),
            pltpu.VMEM((IDXC,), jnp.int32),
            pltpu.VMEM((NBINS_PAD,), jnp.int32),
            pltpu.VMEM((NBINS_PAD,), jnp.int32),
            pltpu.VMEM((RING, EMB_DIM), jnp.float32),
            pltpu.SemaphoreType.DMA,
            pltpu.SemaphoreType.DMA,
        ],
    )
    du, di, dj = scan(batch_user, batch_pos_item, batch_neg_item,
                      user_emb_t, item_emb_t, user_tail, item_tail)
    dot = pl.kernel(
        _dot_body,
        out_type=[jax.ShapeDtypeStruct((BATCH,), jnp.float32)] * 2,
        mesh=mesh,
        compiler_params=cparams,
        scratch_types=[
            pltpu.VMEM((B_PER_W * EMB_DIM,), jnp.float32),
            pltpu.VMEM((B_PER_W * EMB_DIM,), jnp.float32),
            pltpu.VMEM((B_PER_W * EMB_DIM,), jnp.float32),
            pltpu.VMEM((LANES, LANES), jnp.float32),
            pltpu.VMEM((LANES, LANES), jnp.float32),
            pltpu.VMEM((B_PER_W,), jnp.float32),
            pltpu.VMEM((B_PER_W,), jnp.float32),
        ],
    )
    return dot(du, di, dj)


def kernel(batch_user, batch_pos_item, batch_neg_item, user_emb, item_emb):
    ut = user_emb.T
    it = item_emb.T
    pos, neg = _bpr_scores(batch_user.astype(jnp.int32),
                           batch_pos_item.astype(jnp.int32),
                           batch_neg_item.astype(jnp.int32),
                           ut, it,
                           ut[:, TAILP0:],
                           it[:, TAILP0:])
    return (pos[:, None], neg[:, None])


# speedup vs baseline: 1.0833x; 1.0746x over previous
import jax
import jax.numpy as jnp
from jax import lax
from jax.experimental import pallas as pl
from jax.experimental.pallas import tpu as pltpu
from jax.experimental.pallas import tpu_sc as plsc

NUM_CORES = 2
NUM_SUBCORES = 16
LANES = 16
NUM_WORKERS = NUM_CORES * NUM_SUBCORES

BATCH = 16384
EMB_DIM = 64
NROW = 1000000
RANGE = 31232
CHUNK = 256
N_CHUNK = RANGE // CHUNK
TAIL0 = NUM_WORKERS * RANGE
TAIL_MAIN = 512
TAIL_PATCH = 128
TAILP0 = NROW - TAIL_PATCH
LIST_CAP = 2 * BATCH + 2 * LANES
IDXC = 2048
B_PER_W = BATCH // NUM_WORKERS
RING = 32
GROUPS = B_PER_W // LANES
NBINS_PAD = 144


def _scan_body(bu_hbm, bpi_hbm, bni_hbm, ut_hbm, it_hbm, utail_hbm, itail_hbm,
               du_hbm, di_hbm, dj_hbm,
               lst, lst2, bufa, bufb, bufc, tailbuf, idxc, hist, offs,
               stag, ssem, hsem):
    wid = lax.axis_index("s") * NUM_CORES + lax.axis_index("c")
    lo = wid * RANGE
    is_last = wid == NUM_WORKERS - 1
    hi = jnp.where(is_last, NROW, lo + RANGE)

    def build(arr_hbm, tag):
        def chunk_body(ci, cnt):
            pltpu.sync_copy(arr_hbm.at[pl.ds(ci * IDXC, IDXC)], idxc)

            def vec_body(v, cnt):
                r = idxc[pl.ds(v * LANES, LANES)]
                m = (r >= lo) & (r < hi)
                b = ci * IDXC + v * LANES + lax.iota(jnp.int32, LANES)
                entry = ((r - lo) << 16) | (tag << 14) | b
                plsc.store_compressed(lst.at[pl.ds(cnt, LANES)], entry,
                                      mask=m)
                n = plsc.all_reduce_population_count(m)
                return cnt + n[0]

            return lax.fori_loop(0, IDXC // LANES, vec_body, cnt,
                                 unroll=False)

        return chunk_body

    def build_list(arrs):
        cnt = 0
        for arr, tag in arrs:
            cnt = lax.fori_loop(0, BATCH // IDXC, build(arr, tag), cnt,
                                unroll=False)
        return cnt

    def drain_one():
        pltpu.make_async_copy(du_hbm.at[pl.ds(0, EMB_DIM)],
                              stag.at[0], hsem).wait()

    lanes_iota = lax.iota(jnp.int32, LANES)
    lane0 = lanes_iota == 0

    def bucketize(cnt):
        zero = jnp.zeros((LANES,), jnp.int32)
        for z in range(NBINS_PAD // LANES):
            hist[pl.ds(z * LANES, LANES)] = zero

        ones = jnp.ones((LANES,), jnp.int32)

        def count_body(v, carry):
            e16 = lst[pl.ds(v * LANES, LANES)]
            lanes = v * LANES + lanes_iota
            plsc.addupdate_scatter(hist, [e16 >> 24], ones,
                                   mask=lanes < cnt)
            return carry

        lax.fori_loop(0, (cnt + LANES - 1) // LANES, count_body, 0,
                      unroll=False)

        tot = 0
        for g in range(NBINS_PAD // LANES):
            v = hist[pl.ds(g * LANES, LANES)]
            c = plsc.cumsum(v)
            excl = tot + c - v
            offs[pl.ds(g * LANES, LANES)] = excl
            hist[pl.ds(g * LANES, LANES)] = excl
            tot = tot + c[LANES - 1]

        def place_body(p, carry):
            ev = lst[pl.ds(p, LANES)]
            el = ev[0]
            bn = el >> 24
            ov = offs[pl.ds(bn, LANES)]
            o = ov[0]
            offs[pl.ds(bn, LANES)] = jnp.where(lane0, ov + 1, ov)
            plsc.store_compressed(lst2.at[pl.ds(o, LANES)], ev, mask=lane0)
            return carry

        lax.fori_loop(0, cnt, place_body, 0, unroll=False)

    def process(buf, c0, pass_user, fired0, bin_override=None):
        bn = c0 >> 8 if bin_override is None else bin_override
        s = hist[pl.ds(bn, LANES)][0]
        epos = offs[pl.ds(bn, LANES)][0]

        def ent_body(p, fired):
            el = lst2[pl.ds(p, LANES)][0]
            slot = fired & (RING - 1)

            @pl.when(fired >= RING)
            def _():
                drain_one()

            cl = (el >> 16) - c0
            b = el & (BATCH - 1)
            clv = jnp.full((LANES,), cl, jnp.int32)
            for v4 in range(EMB_DIM // LANES):
                g = plsc.load_gather(buf, [lanes_iota + v4 * LANES, clv])
                stag[slot, pl.ds(v4 * LANES, LANES)] = g
            dst = pl.ds(b * EMB_DIM, EMB_DIM)
            src = stag.at[slot]
            if pass_user:
                pltpu.async_copy(src, du_hbm.at[dst], hsem)
            else:
                tl = (el >> 14) & 3
                @pl.when(tl == 1)
                def _():
                    pltpu.async_copy(src, di_hbm.at[dst], hsem)
                @pl.when(tl == 2)
                def _():
                    pltpu.async_copy(src, dj_hbm.at[dst], hsem)

            return fired + 1

        return lax.fori_loop(s, epos, ent_body, fired0, unroll=False)

    def prestart(tab_hbm):
        for c8 in range(EMB_DIM // 8):
            pltpu.async_copy(
                tab_hbm.at[pl.ds(c8 * 8, 8), pl.ds(lo, CHUNK)],
                bufa.at[pl.ds(c8 * 8, 8)], ssem)
        for c8 in range(EMB_DIM // 8):
            pltpu.async_copy(
                tab_hbm.at[pl.ds(c8 * 8, 8), pl.ds(lo + CHUNK, CHUNK)],
                bufb.at[pl.ds(c8 * 8, 8)], ssem)

    def stream_range(tab_hbm, tail_hbm, pass_user):
        def start(k, buf):
            for c8 in range(EMB_DIM // 8):
                pltpu.async_copy(
                    tab_hbm.at[pl.ds(c8 * 8, 8), pl.ds(lo + k * CHUNK, CHUNK)],
                    buf.at[pl.ds(c8 * 8, 8)], ssem)

        def start_if(k, buf):
            @pl.when(k < N_CHUNK)
            def _():
                start(k, buf)

        def wait(buf):
            pltpu.make_async_copy(tab_hbm.at[:, pl.ds(0, CHUNK)], buf,
                                  ssem).wait()

        def tri_body(p, fired):
            k = p * 3
            wait(bufa)
            start_if(k + 2, bufc)
            fired = process(bufa, k * CHUNK, pass_user, fired)
            wait(bufb)
            start_if(k + 3, bufa)
            fired = process(bufb, (k + 1) * CHUNK, pass_user, fired)
            wait(bufc)
            start_if(k + 4, bufb)
            fired = process(bufc, (k + 2) * CHUNK, pass_user, fired)
            return fired

        fired = lax.fori_loop(0, N_CHUNK // 3, tri_body, 0, unroll=False)
        wait(bufa)
        fired = process(bufa, (N_CHUNK - 2) * CHUNK, pass_user, fired)
        wait(bufb)
        fired = process(bufb, (N_CHUNK - 1) * CHUNK, pass_user, fired)

        def drain_n(n):
            def drain_body(d, carry):
                drain_one()
                return carry

            lax.fori_loop(0, jnp.minimum(n, RING), drain_body, 0,
                          unroll=False)

        @pl.when(is_last)
        def _():
            f2 = fired
            for t in range(TAIL_MAIN // CHUNK):
                pltpu.sync_copy(
                    tab_hbm.at[:, pl.ds(TAIL0 + t * CHUNK, CHUNK)], bufa)
                f2 = process(bufa, TAIL0 - lo + t * CHUNK, pass_user, f2)
            pltpu.sync_copy(tail_hbm, tailbuf)
            f3 = process(tailbuf, TAILP0 - lo, pass_user, f2,
                         bin_override=(TAIL0 - lo + TAIL_MAIN) >> 8)
            drain_n(f3)

        @pl.when(jnp.logical_not(is_last))
        def _():
            drain_n(fired)

    prestart(ut_hbm)
    cnt_u = build_list(((bu_hbm, 0),))
    bucketize(cnt_u)
    stream_range(ut_hbm, utail_hbm, True)
    prestart(it_hbm)
    cnt_i = build_list(((bpi_hbm, 1), (bni_hbm, 2)))
    bucketize(cnt_i)
    stream_range(it_hbm, itail_hbm, False)


def _dot_body(du_hbm, di_hbm, dj_hbm, pos_hbm, neg_hbm,
              ru, ri, rj, accp_s, accn_s, pos_v, neg_v):
    wid = lax.axis_index("s") * NUM_CORES + lax.axis_index("c")
    base = wid * B_PER_W
    nwords = B_PER_W * EMB_DIM
    pltpu.sync_copy(du_hbm.at[pl.ds(base * EMB_DIM, nwords)], ru)
    pltpu.sync_copy(di_hbm.at[pl.ds(base * EMB_DIM, nwords)], ri)
    pltpu.sync_copy(dj_hbm.at[pl.ds(base * EMB_DIM, nwords)], rj)
    lanes_iota = lax.iota(jnp.int32, LANES)

    def group_body(g, carry):
        for row_l in range(LANES):
            off = (g * LANES + row_l) * EMB_DIM
            accp = jnp.zeros((LANES,), jnp.float32)
            accn = jnp.zeros((LANES,), jnp.float32)
            for v in range(EMB_DIM // LANES):
                sl = pl.ds(off + v * LANES, LANES)
                u = ru[sl]
                iv = ri[sl]
                jv = rj[sl]
                accp = accp + u * iv
                accn = accn + u * jv
            accp_s[row_l] = accp
            accn_s[row_l] = accn
        sump = jnp.zeros((LANES,), jnp.float32)
        sumn = jnp.zeros((LANES,), jnp.float32)
        for l in range(LANES):
            col = jnp.full((LANES,), l, jnp.int32)
            sump = sump + plsc.load_gather(accp_s, [lanes_iota, col])
            sumn = sumn + plsc.load_gather(accn_s, [lanes_iota, col])
        out = pl.ds(g * LANES, LANES)
        pos_v[out] = sump
        neg_v[out] = sumn
        return carry

    lax.fori_loop(0, GROUPS, group_body, 0, unroll=False)
    pltpu.sync_copy(pos_v, pos_hbm.at[pl.ds(base, B_PER_W)])
    pltpu.sync_copy(neg_v, neg_hbm.at[pl.ds(base, B_PER_W)])


@jax.jit
def _bpr_scores(batch_user, batch_pos_item, batch_neg_item,
                user_emb_t, item_emb_t, user_tail, item_tail):
    mesh = plsc.VectorSubcoreMesh(core_axis_name="c", subcore_axis_name="s",
                                  num_cores=NUM_CORES,
                                  num_subcores=NUM_SUBCORES)
    cparams = pltpu.CompilerParams(needs_layout_passes=False,
                                   use_tc_tiling_on_sc=True)
    scan = pl.kernel(
        _scan_body,
        out_type=[jax.ShapeDtypeStruct((BATCH * EMB_DIM,), jnp.float32)] * 3,
        mesh=mesh,
        compiler_params=cparams,
        scratch_types=[
            pltpu.VMEM((LIST_CAP,), jnp.int32),
            pltpu.VMEM((LIST_CAP,), jnp.int32),
            pltpu.VMEM((EMB_DIM, CHUNK), jnp.float32),
            pltpu.VMEM((EMB_DIM, CHUNK), jnp.float32),
            pltpu.VMEM((EMB_DIM, CHUNK), jnp.float32),
            pltpu.VMEM((EMB_DIM, TAIL_PATCH), jnp.float32),
            pltpu.VMEM((IDXC,), jnp.int32),
            pltpu.VMEM((NBINS_PAD,), jnp.int32),
            pltpu.VMEM((NBINS_PAD,), jnp.int32),
            pltpu.VMEM((RING, EMB_DIM), jnp.float32),
            pltpu.SemaphoreType.DMA,
            pltpu.SemaphoreType.DMA,
        ],
    )
    du, di, dj = scan(batch_user, batch_pos_item, batch_neg_item,
                      user_emb_t, item_emb_t, user_tail, item_tail)
    dot = pl.kernel(
        _dot_body,
        out_type=[jax.ShapeDtypeStruct((BATCH,), jnp.float32)] * 2,
        mesh=mesh,
        compiler_params=cparams,
        scratch_types=[
            pltpu.VMEM((B_PER_W * EMB_DIM,), jnp.float32),
            pltpu.VMEM((B_PER_W * EMB_DIM,), jnp.float32),
            pltpu.VMEM((B_PER_W * EMB_DIM,), jnp.float32),
            pltpu.VMEM((LANES, LANES), jnp.float32),
            pltpu.VMEM((LANES, LANES), jnp.float32),
            pltpu.VMEM((B_PER_W,), jnp.float32),
            pltpu.VMEM((B_PER_W,), jnp.float32),
        ],
    )
    return dot(du, di, dj)


def kernel(batch_user, batch_pos_item, batch_neg_item, user_emb, item_emb):
    ut = user_emb.T
    it = item_emb.T
    pos, neg = _bpr_scores(batch_user.astype(jnp.int32),
                           batch_pos_item.astype(jnp.int32),
                           batch_neg_item.astype(jnp.int32),
                           ut, it,
                           ut[:, TAILP0:],
                           it[:, TAILP0:])
    return (pos[:, None], neg[:, None])
